# baseline (speedup 1.0000x reference)
import functools
import jax
import jax.numpy as jnp
from jax import lax
from jax.experimental import pallas as pl
from jax.experimental.pallas import tpu as pltpu
from jax.experimental.pallas import tpu_sc as plsc

BATCH = 4
SEQ_LEN = 8192
FEAT_DIM = 768
N = SEQ_LEN * FEAT_DIM
NW = 32
W = N // NW
BLK = 24576
NBLK = W // BLK
LANES = 16


def _body(in_hbm, pos_hbm, time_hbm, out_hbm, cbuf, tbuf, ibuf):
    wid = lax.axis_index("s") * 2 + lax.axis_index("c")
    base = wid * W

    for k in range(NBLK):
        off = base + k * BLK
        pltpu.sync_copy(pos_hbm.at[pl.ds(off, BLK)], cbuf)
        pltpu.sync_copy(time_hbm.at[pl.ds(off, BLK)], tbuf)

        @pl.loop(0, BLK // LANES)
        def _combine(i):
            s = pl.ds(i * LANES, LANES)
            cbuf[s] = cbuf[s] + tbuf[s]

        for b in range(BATCH):
            pltpu.sync_copy(in_hbm.at[b, pl.ds(off, BLK)], ibuf)

            @pl.loop(0, BLK // LANES)
            def _accum(i):
                s = pl.ds(i * LANES, LANES)
                ibuf[s] = ibuf[s] + cbuf[s]

            pltpu.sync_copy(ibuf, out_hbm.at[b, pl.ds(off, BLK)])


@jax.jit
def kernel(inputs, pos_table, time_table):
    in_flat = inputs.reshape(BATCH, N)
    pos_flat = pos_table.reshape(N)
    time_flat = time_table.reshape(N)

    mesh = plsc.VectorSubcoreMesh(core_axis_name="c", subcore_axis_name="s")
    out = pl.kernel(
        _body,
        out_type=jax.ShapeDtypeStruct((BATCH, N), jnp.float32),
        mesh=mesh,
        scratch_types=[
            pltpu.VMEM((BLK,), jnp.float32),
            pltpu.VMEM((BLK,), jnp.float32),
            pltpu.VMEM((BLK,), jnp.float32),
        ],
    )(in_flat, pos_flat, time_flat)
    return out.reshape(BATCH, SEQ_LEN, FEAT_DIM)


# speedup vs baseline: 1.8907x; 1.8907x over previous
import jax
import jax.numpy as jnp
from jax import lax
from jax.experimental import pallas as pl
from jax.experimental.pallas import tpu as pltpu
from jax.experimental.pallas import tpu_sc as plsc

BATCH = 4
SEQ_LEN = 8192
FEAT_DIM = 768
N = SEQ_LEN * FEAT_DIM
NW = 32
W = N // NW
BLK = 8192
NBLK = W // BLK
LANES = 16
UNROLL = 8


def _body(in_hbm, pos_hbm, time_hbm, out_hbm, *scr):
    pbuf = scr[0:2]
    tbuf = scr[2:4]
    ibuf = scr[4:12]
    psem = scr[12:14]
    tsem = scr[14:16]
    isem = scr[16:20]
    osem = scr[20:28]

    wid = lax.axis_index("s") * 2 + lax.axis_index("c")
    base = wid * W

    def wait_in(sem, vref, hbm_ref):
        pltpu.make_async_copy(hbm_ref.at[pl.ds(0, BLK)], vref, sem).wait()

    def wait_out(slot, b):
        pltpu.make_async_copy(
            ibuf[slot], out_hbm.at[b, pl.ds(0, BLK)], osem[slot]
        ).wait()

    def do_block(k, p, prefetch, wait_prev_out):
        pn = 1 - p
        off = base + k * BLK
        wait_in(psem[p], pbuf[p], pos_hbm)
        wait_in(tsem[p], tbuf[p], time_hbm)
        if prefetch:
            offn = off + BLK
            pltpu.async_copy(pos_hbm.at[pl.ds(offn, BLK)], pbuf[pn], psem[pn])
            pltpu.async_copy(time_hbm.at[pl.ds(offn, BLK)], tbuf[pn], tsem[pn])

        pb = pbuf[p]
        tb = tbuf[p]

        @pl.loop(0, BLK // LANES, unroll=UNROLL)
        def _combine(i):
            s = pl.ds(i * LANES, LANES)
            pb[s] = pb[s] + tb[s]

        for b in range(BATCH):
            slot = 2 * b + p
            ib = ibuf[slot]
            wait_in(isem[b], ib, pos_hbm)

            @pl.loop(0, BLK // LANES, unroll=UNROLL)
            def _accum(i):
                s = pl.ds(i * LANES, LANES)
                plsc.addupdate(ib.at[s], pb[s])

            pltpu.async_copy(ib, out_hbm.at[b, pl.ds(off, BLK)], osem[slot])
            if prefetch:
                nslot = 2 * b + pn
                if wait_prev_out:
                    wait_out(nslot, b)
                pltpu.async_copy(
                    in_hbm.at[b, pl.ds(off + BLK, BLK)], ibuf[nslot], isem[b]
                )

    off0 = base
    pltpu.async_copy(pos_hbm.at[pl.ds(off0, BLK)], pbuf[0], psem[0])
    pltpu.async_copy(time_hbm.at[pl.ds(off0, BLK)], tbuf[0], tsem[0])
    for b in range(BATCH):
        pltpu.async_copy(in_hbm.at[b, pl.ds(off0, BLK)], ibuf[2 * b], isem[b])

    do_block(0, 0, prefetch=True, wait_prev_out=False)

    @pl.loop(1, NBLK - 1, step=2)
    def _mid(k0):
        do_block(k0, 1, prefetch=True, wait_prev_out=True)
        do_block(k0 + 1, 0, prefetch=True, wait_prev_out=True)

    do_block(NBLK - 1, 1, prefetch=False, wait_prev_out=False)

    for b in range(BATCH):
        wait_out(2 * b, b)
        wait_out(2 * b + 1, b)


@jax.jit
def kernel(inputs, pos_table, time_table):
    in_flat = inputs.reshape(BATCH, N)
    pos_flat = pos_table.reshape(N)
    time_flat = time_table.reshape(N)

    mesh = plsc.VectorSubcoreMesh(core_axis_name="c", subcore_axis_name="s")
    out = pl.kernel(
        _body,
        out_type=jax.ShapeDtypeStruct((BATCH, N), jnp.float32),
        mesh=mesh,
        scratch_types=(
            [pltpu.VMEM((BLK,), jnp.float32) for _ in range(12)]
            + [pltpu.SemaphoreType.DMA for _ in range(16)]
        ),
    )(in_flat, pos_flat, time_flat)
    return out.reshape(BATCH, SEQ_LEN, FEAT_DIM)


# speedup vs baseline: 4.1004x; 2.1687x over previous
import jax
import jax.numpy as jnp
from jax import lax
from jax.experimental import pallas as pl
from jax.experimental.pallas import tpu as pltpu
from jax.experimental.pallas import tpu_sc as plsc

BATCH = 4
SEQ_LEN = 8192
FEAT_DIM = 768
NW = 32
ROWS_W = SEQ_LEN // NW
R = 8
NBLK = ROWS_W // R
LANES = 16
CGRP = FEAT_DIM // LANES
UNROLL = 8


def _body(in_hbm, pos_hbm, time_hbm, out_hbm, *scr):
    pbuf = scr[0:2]
    tbuf = scr[2:4]
    ibuf = scr[4:12]
    psem = scr[12:14]
    tsem = scr[14:16]
    isem = scr[16:20]
    osem = scr[20:28]

    wid = lax.axis_index("s") * 2 + lax.axis_index("c")
    base = wid * ROWS_W

    def wait_in(sem, vref):
        pltpu.make_async_copy(pos_hbm.at[pl.ds(0, R), :], vref, sem).wait()

    def wait_out(slot, b):
        pltpu.make_async_copy(
            ibuf[slot], out_hbm.at[b, pl.ds(0, R), :], osem[slot]
        ).wait()

    def vloop(body):
        @pl.loop(0, R)
        def _row(r):
            @pl.loop(0, CGRP, unroll=UNROLL)
            def _col(c):
                body(r, pl.ds(c * LANES, LANES))

    def do_block(k, p, prefetch, wait_prev_out):
        pn = 1 - p
        roff = base + k * R
        wait_in(psem[p], pbuf[p])
        wait_in(tsem[p], tbuf[p])
        if prefetch:
            roffn = roff + R
            pltpu.async_copy(pos_hbm.at[pl.ds(roffn, R), :], pbuf[pn], psem[pn])
            pltpu.async_copy(time_hbm.at[pl.ds(roffn, R), :], tbuf[pn], tsem[pn])

        pb = pbuf[p]
        tb = tbuf[p]

        def _combine(r, s):
            pb[r, s] = pb[r, s] + tb[r, s]

        vloop(_combine)

        for b in range(BATCH):
            slot = 2 * b + p
            ib = ibuf[slot]
            wait_in(isem[b], ib)

            def _accum(r, s, ib=ib):
                plsc.addupdate(ib.at[r, s], pb[r, s])

            vloop(_accum)

            pltpu.async_copy(ib, out_hbm.at[b, pl.ds(roff, R), :], osem[slot])
            if prefetch:
                nslot = 2 * b + pn
                if wait_prev_out:
                    wait_out(nslot, b)
                pltpu.async_copy(
                    in_hbm.at[b, pl.ds(roff + R, R), :], ibuf[nslot], isem[b]
                )

    pltpu.async_copy(pos_hbm.at[pl.ds(base, R), :], pbuf[0], psem[0])
    pltpu.async_copy(time_hbm.at[pl.ds(base, R), :], tbuf[0], tsem[0])
    for b in range(BATCH):
        pltpu.async_copy(in_hbm.at[b, pl.ds(base, R), :], ibuf[2 * b], isem[b])

    do_block(0, 0, prefetch=True, wait_prev_out=False)

    @pl.loop(1, NBLK - 1, step=2)
    def _mid(k0):
        do_block(k0, 1, prefetch=True, wait_prev_out=True)
        do_block(k0 + 1, 0, prefetch=True, wait_prev_out=True)

    do_block(NBLK - 1, 1, prefetch=False, wait_prev_out=False)

    for b in range(BATCH):
        wait_out(2 * b, b)
        wait_out(2 * b + 1, b)


@jax.jit
def kernel(inputs, pos_table, time_table):
    mesh = plsc.VectorSubcoreMesh(core_axis_name="c", subcore_axis_name="s")
    return pl.kernel(
        _body,
        out_type=jax.ShapeDtypeStruct((BATCH, SEQ_LEN, FEAT_DIM), jnp.float32),
        mesh=mesh,
        compiler_params=pltpu.CompilerParams(use_tc_tiling_on_sc=True),
        scratch_types=(
            [pltpu.VMEM((R, FEAT_DIM), jnp.float32) for _ in range(12)]
            + [pltpu.SemaphoreType.DMA for _ in range(16)]
        ),
    )(inputs, pos_table, time_table)
